# baseline (speedup 1.0000x reference)
import jax
import jax.numpy as jnp
from jax import lax
from jax.experimental import pallas as pl
from jax.experimental.pallas import tpu as pltpu
from jax.experimental.pallas import tpu_sc as plsc

N_USERS = 50000
N_ITEMS = 50000
N_NODES = N_USERS + 1 + N_ITEMS + 1
EMB_DIM = 64
N_EDGES = 1000000
BATCH = 4096
REG_WEIGHT = 1e-4

LANES = 16
NW = 32
EBLK = 128
E_PER_W = 31744
BLK_PER_W = E_PER_W // EBLK
E_PAD = NW * E_PER_W
CH_BLKS = 31
N_CH = BLK_PER_W // CH_BLKS
CH_E = CH_BLKS * EBLK
MAP_ROWS = 100352
NSLOT = 3 * BATCH
SLOT_ROWS = 12544
TRASH_SLOT = NSLOT
SROWS_PER_TILE = SLOT_ROWS // 16
DST_PAD = N_NODES
NCHUNK = 4
STRIP = 31


def _filter_body(src_hbm, dst_hbm, map_hbm, chbm, cnt_hbm,
                 mp, dqc, sqc, cbuf, cst):
    ci = lax.axis_index("c")
    si = lax.axis_index("s")
    wid = si * 2 + ci
    tblk = wid * BLK_PER_W

    pltpu.sync_copy(map_hbm, mp)

    trash16 = jnp.full((LANES,), TRASH_SLOT, jnp.int32)
    nflushed = jnp.int32(0)
    for ch in range(N_CH):
        pltpu.sync_copy(dst_hbm.at[wid, pl.ds(ch * CH_E, CH_E)], dqc)
        pltpu.sync_copy(src_hbm.at[wid, pl.ds(ch * CH_E, CH_E)], sqc)

        def blk(j, cur):
            for g in range(EBLK // LANES):
                d16 = dqc[pl.ds(j * EBLK + g * LANES, LANES)]
                s16 = sqc[pl.ds(j * EBLK + g * LANES, LANES)]
                slot16 = plsc.load_gather(mp, [d16])
                m = slot16 != TRASH_SLOT
                packed = s16 * 16384 + slot16
                plsc.store_compressed(cbuf.at[pl.ds(cur, LANES)], packed, mask=m)
                cur = cur + jnp.sum(m.astype(jnp.int32))
            return cur

        cur = lax.fori_loop(0, CH_BLKS, blk, jnp.int32(0))
        tgt = ((cur + EBLK - 1) // EBLK) * EBLK
        for g in range(EBLK // LANES):
            @pl.when(cur + g * LANES < tgt)
            def _():
                cbuf[pl.ds(cur + g * LANES, LANES)] = trash16

        nb = tgt // EBLK

        def flush(b, _):
            pltpu.sync_copy(cbuf.at[pl.ds(b * EBLK, EBLK)],
                            chbm.at[pl.ds((tblk + nflushed + b) * EBLK, EBLK)])
            return _

        lax.fori_loop(0, nb, flush, jnp.int32(0))
        nflushed = nflushed + nb

    for g in range(EBLK // LANES):
        cbuf[pl.ds(g * LANES, LANES)] = trash16
    tpad = ((nflushed + STRIP - 1) // STRIP) * STRIP

    def padb(b, _):
        pltpu.sync_copy(cbuf.at[pl.ds(0, EBLK)],
                        chbm.at[pl.ds((tblk + nflushed + b) * EBLK, EBLK)])
        return _

    lax.fori_loop(0, tpad - nflushed, padb, jnp.int32(0))
    cst[pl.ds(0, LANES)] = jnp.full((LANES,), 1, jnp.int32) * tpad
    pltpu.sync_copy(cst, cnt_hbm.at[pl.ds(wid * LANES, LANES)])


def _reduce_body(x4_hbm, chbm, cnt_hbm, zrow_hbm, out_hbm,
                 pbl, gist, sst, rows, cntv, zdbuf, acc, sem, sem2):
    ci = lax.axis_index("c")
    si = lax.axis_index("s")
    wid = si * 2 + ci
    tblk = wid * BLK_PER_W

    pltpu.sync_copy(cnt_hbm.at[pl.ds(wid * LANES, LANES)], cntv)
    nblocks = cntv[pl.ds(0, LANES)][0]
    nstrips = nblocks // STRIP

    for c in range(NCHUNK):
        pltpu.sync_copy(zrow_hbm, zdbuf)
        pltpu.sync_copy(zdbuf, acc.at[pl.ds(si * SROWS_PER_TILE, SROWS_PER_TILE)])
        plsc.subcore_barrier()

        def strip_fn(st, _):
            base = (tblk + st * STRIP) * EBLK
            pltpu.sync_copy(chbm.at[pl.ds(base, STRIP * EBLK)], pbl)
            for j in range(STRIP):
                for g in range(EBLK // LANES):
                    p16 = pbl[pl.ds(j * EBLK + g * LANES, LANES)]
                    slot16 = jnp.bitwise_and(p16, 16383)
                    src16 = lax.shift_right_logical(p16, 14)
                    gist[j, pl.ds(g * LANES, LANES)] = src16 * NCHUNK + c
                    sst[j, pl.ds(g * LANES, LANES)] = slot16
            for lo in range(0, STRIP, 8):
                hi = min(lo + 8, STRIP)
                gh = [pltpu.async_copy(x4_hbm.at[gist.at[j]], rows.at[j], sem)
                      for j in range(lo, hi)]
                sh = []
                for k, j in enumerate(range(lo, hi)):
                    gh[k].wait()
                    sh.append(pltpu.async_copy(rows.at[j], acc.at[sst.at[j]],
                                               sem2, add=True))
                for h in sh:
                    h.wait()
            return _

        lax.fori_loop(0, nstrips, strip_fn, jnp.int32(0))
        plsc.subcore_barrier()

        r = si * SROWS_PER_TILE
        pltpu.sync_copy(acc.at[pl.ds(r, SROWS_PER_TILE)], zdbuf)
        pltpu.sync_copy(zdbuf, out_hbm.at[ci, c, pl.ds(r, SROWS_PER_TILE)])
        plsc.subcore_barrier()


@jax.jit
def _gcn_sc(x4, srcw, dstw, slotmap, zrow):
    mesh = plsc.VectorSubcoreMesh(core_axis_name="c", subcore_axis_name="s")
    f1 = pl.kernel(
        _filter_body,
        out_type=(
            jax.ShapeDtypeStruct((E_PAD,), jnp.int32),
            jax.ShapeDtypeStruct((NW * LANES,), jnp.int32),
        ),
        mesh=mesh,
        scratch_types=[
            pltpu.VMEM((MAP_ROWS,), jnp.int32),
            pltpu.VMEM((CH_E,), jnp.int32),
            pltpu.VMEM((CH_E,), jnp.int32),
            pltpu.VMEM((CH_E + 2 * EBLK,), jnp.int32),
            pltpu.VMEM((LANES,), jnp.int32),
        ],
        compiler_params=pltpu.CompilerParams(use_tc_tiling_on_sc=False,
                                             needs_layout_passes=False),
    )
    chbm, cnts = f1(srcw, dstw, slotmap)

    f2 = pl.kernel(
        _reduce_body,
        out_type=jax.ShapeDtypeStruct((2, NCHUNK, SLOT_ROWS, LANES), jnp.float32),
        mesh=mesh,
        scratch_types=[
            pltpu.VMEM((STRIP * EBLK,), jnp.int32),
            pltpu.VMEM((STRIP, EBLK), jnp.int32),
            pltpu.VMEM((STRIP, EBLK), jnp.int32),
            pltpu.VMEM((STRIP, EBLK, LANES), jnp.float32),
            pltpu.VMEM((LANES,), jnp.int32),
            pltpu.VMEM((SROWS_PER_TILE, LANES), jnp.float32),
            pltpu.VMEM_SHARED((SLOT_ROWS, LANES), jnp.float32),
            pltpu.SemaphoreType.DMA,
            pltpu.SemaphoreType.DMA,
        ],
        compiler_params=pltpu.CompilerParams(use_tc_tiling_on_sc=False,
                                             needs_layout_passes=False),
    )
    return f2(x4, chbm, cnts, zrow)


def kernel(batch_data, edge_index, user_embedding, item_embedding, W_den, W_os,
           conv_w, last_stage, degree_new, degree_old):
    total_embeddings = jnp.concatenate([user_embedding, item_embedding], axis=0)

    x_den = jax.nn.relu(degree_old @ W_den.T) + degree_new
    degree_den = jnp.sqrt(x_den)
    norm_degree = (1.0 / (degree_den + 1e-9)).reshape(-1)

    users = batch_data[:, 0, 0]
    items = batch_data[:, 0, 1:]
    nodes = jnp.concatenate([users, (N_USERS + 1 + items).reshape(-1)])
    slotmap = jnp.full((MAP_ROWS,), TRASH_SLOT, jnp.int32)
    slotmap = slotmap.at[nodes].set(jnp.arange(NSLOT, dtype=jnp.int32))
    slots_b = slotmap[nodes]

    x1 = norm_degree[:, None] * total_embeddings
    x4 = x1.reshape(N_NODES * NCHUNK, LANES)

    dst = edge_index[0]
    src = edge_index[1]
    pad = E_PAD - N_EDGES
    srcw = jnp.concatenate([src, jnp.zeros((pad,), jnp.int32)]).reshape(NW, E_PER_W)
    dstw = jnp.concatenate([dst, jnp.full((pad,), DST_PAD, jnp.int32)]).reshape(NW, E_PER_W)
    zrow = jnp.zeros((SROWS_PER_TILE, LANES), jnp.float32)

    part = _gcn_sc(x4, srcw, dstw, slotmap, zrow)
    summed = (part[0] + part[1])[:, :NSLOT]
    emb1_slots = summed.transpose(1, 0, 2).reshape(NSLOT, EMB_DIM)

    rscale_vec = jnp.sqrt(jax.nn.relu(degree_old @ W_os.T)) / (degree_den + 1e-9)

    sc_nodes = jnp.stack([norm_degree, rscale_vec.reshape(-1)], axis=1)
    sc_b = sc_nodes[nodes]
    nd_b = sc_b[:, 0:1]
    rs_b = sc_b[:, 1:2]
    emb1_b = nd_b * emb1_slots[slots_b]
    te_b = total_embeddings[nodes]
    ls_b = jnp.take(last_stage, nodes, axis=1)

    fused0 = conv_w[0, 0] * (ls_b[0] * rs_b) + conv_w[0, 1] * te_b
    fused1 = conv_w[1, 0] * (ls_b[1] * rs_b) + conv_w[1, 1] * emb1_b
    layer_b = (te_b + fused0 + fused1) / 3.0
    nrm = jnp.linalg.norm(layer_b, axis=-1, keepdims=True)
    layer_b = layer_b / jnp.maximum(nrm, 1e-12)
    total2_b = layer_b + te_b

    user_feature = total2_b[:BATCH][:, None, :]
    item_feature = total2_b[BATCH:].reshape(BATCH, 2, EMB_DIM)
    scores = jnp.sum(user_feature * item_feature, axis=2)
    bpr = -jnp.mean(jax.nn.log_sigmoid(scores[:, 0] - scores[:, 1]))

    emb_loss = (jnp.linalg.norm(user_embedding) + jnp.linalg.norm(item_embedding)) / item_embedding.shape[0]
    return bpr + REG_WEIGHT * emb_loss


# speedup vs baseline: 2.8618x; 2.8618x over previous
import jax
import jax.numpy as jnp
from jax import lax
from jax.experimental import pallas as pl
from jax.experimental.pallas import tpu as pltpu
from jax.experimental.pallas import tpu_sc as plsc

N_USERS = 50000
N_ITEMS = 50000
N_NODES = N_USERS + 1 + N_ITEMS + 1
EMB_DIM = 64
N_EDGES = 1000000
BATCH = 4096
REG_WEIGHT = 1e-4

LANES = 16
NW = 32
EBLK = 128
E_PER_W = 31744
BLK_PER_W = E_PER_W // EBLK
E_PAD = NW * E_PER_W
CH_BLKS = 31
N_CH = BLK_PER_W // CH_BLKS
CH_E = CH_BLKS * EBLK
MAP_ROWS = 100352
NSLOT = 3 * BATCH
SLOT_ROWS = 12544
TRASH_SLOT = NSLOT
SROWS_PER_TILE = SLOT_ROWS // 16
DST_PAD = N_NODES
NCHUNK = 4
STRIP = 8


def _filter_body(src_hbm, dst_hbm, map_hbm, chbm, cnt_hbm,
                 mp, dqc, sqc, cbuf, cst):
    ci = lax.axis_index("c")
    si = lax.axis_index("s")
    wid = si * 2 + ci
    tblk = wid * BLK_PER_W

    pltpu.sync_copy(map_hbm, mp)

    trash16 = jnp.full((LANES,), TRASH_SLOT, jnp.int32)
    nflushed = jnp.int32(0)
    for ch in range(N_CH):
        pltpu.sync_copy(dst_hbm.at[wid, pl.ds(ch * CH_E, CH_E)], dqc)
        pltpu.sync_copy(src_hbm.at[wid, pl.ds(ch * CH_E, CH_E)], sqc)

        def blk(j, cur):
            for g in range(EBLK // LANES):
                d16 = dqc[pl.ds(j * EBLK + g * LANES, LANES)]
                s16 = sqc[pl.ds(j * EBLK + g * LANES, LANES)]
                slot16 = plsc.load_gather(mp, [d16])
                m = slot16 != TRASH_SLOT
                packed = s16 * 16384 + slot16
                plsc.store_compressed(cbuf.at[pl.ds(cur, LANES)], packed, mask=m)
                cur = cur + jnp.sum(m.astype(jnp.int32))
            return cur

        cur = lax.fori_loop(0, CH_BLKS, blk, jnp.int32(0))
        tgt = ((cur + EBLK - 1) // EBLK) * EBLK
        for g in range(EBLK // LANES):
            @pl.when(cur + g * LANES < tgt)
            def _():
                cbuf[pl.ds(cur + g * LANES, LANES)] = trash16

        nb = tgt // EBLK

        def flush(b, _):
            pltpu.sync_copy(cbuf.at[pl.ds(b * EBLK, EBLK)],
                            chbm.at[pl.ds((tblk + nflushed + b) * EBLK, EBLK)])
            return _

        lax.fori_loop(0, nb, flush, jnp.int32(0))
        nflushed = nflushed + nb

    for g in range(EBLK // LANES):
        cbuf[pl.ds(g * LANES, LANES)] = trash16
    tpad = ((nflushed + STRIP - 1) // STRIP) * STRIP

    def padb(b, _):
        pltpu.sync_copy(cbuf.at[pl.ds(0, EBLK)],
                        chbm.at[pl.ds((tblk + nflushed + b) * EBLK, EBLK)])
        return _

    lax.fori_loop(0, tpad - nflushed, padb, jnp.int32(0))
    cst[pl.ds(0, LANES)] = jnp.full((LANES,), 1, jnp.int32) * tpad
    pltpu.sync_copy(cst, cnt_hbm.at[pl.ds(wid * LANES, LANES)])


def _reduce_body(x4_hbm, chbm, cnt_hbm, zrow_hbm, out_hbm,
                 pbl, gist, sst, rows, cntv, zdbuf, acc, sem, sem2):
    ci = lax.axis_index("c")
    si = lax.axis_index("s")
    wid = si * 2 + ci
    tblk = wid * BLK_PER_W

    pltpu.sync_copy(cnt_hbm.at[pl.ds(wid * LANES, LANES)], cntv)
    nblocks = cntv[pl.ds(0, LANES)][0]
    nstrips = nblocks // STRIP

    for c in range(NCHUNK):
        pltpu.sync_copy(zrow_hbm, zdbuf)
        pltpu.sync_copy(zdbuf, acc.at[pl.ds(si * SROWS_PER_TILE, SROWS_PER_TILE)])
        plsc.subcore_barrier()

        def strip_fn(st, _):
            base = (tblk + st * STRIP) * EBLK
            pltpu.sync_copy(chbm.at[pl.ds(base, STRIP * EBLK)], pbl)
            for j in range(STRIP):
                for g in range(EBLK // LANES):
                    p16 = pbl[pl.ds(j * EBLK + g * LANES, LANES)]
                    slot16 = jnp.bitwise_and(p16, 16383)
                    src16 = lax.shift_right_logical(p16, 14)
                    gist[j, pl.ds(g * LANES, LANES)] = src16 * NCHUNK + c
                    sst[j, pl.ds(g * LANES, LANES)] = slot16
            for lo in range(0, STRIP, 8):
                hi = min(lo + 8, STRIP)
                gh = [pltpu.async_copy(x4_hbm.at[gist.at[j]], rows.at[j], sem)
                      for j in range(lo, hi)]
                sh = []
                for k, j in enumerate(range(lo, hi)):
                    gh[k].wait()
                    sh.append(pltpu.async_copy(rows.at[j], acc.at[sst.at[j]],
                                               sem2, add=True))
                for h in sh:
                    h.wait()
            return _

        lax.fori_loop(0, nstrips, strip_fn, jnp.int32(0))
        plsc.subcore_barrier()

        r = si * SROWS_PER_TILE
        pltpu.sync_copy(acc.at[pl.ds(r, SROWS_PER_TILE)], zdbuf)
        pltpu.sync_copy(zdbuf, out_hbm.at[ci, c, pl.ds(r, SROWS_PER_TILE)])
        plsc.subcore_barrier()


@jax.jit
def _gcn_sc(x4, srcw, dstw, slotmap, zrow):
    mesh = plsc.VectorSubcoreMesh(core_axis_name="c", subcore_axis_name="s")
    f1 = pl.kernel(
        _filter_body,
        out_type=(
            jax.ShapeDtypeStruct((E_PAD,), jnp.int32),
            jax.ShapeDtypeStruct((NW * LANES,), jnp.int32),
        ),
        mesh=mesh,
        scratch_types=[
            pltpu.VMEM((MAP_ROWS,), jnp.int32),
            pltpu.VMEM((CH_E,), jnp.int32),
            pltpu.VMEM((CH_E,), jnp.int32),
            pltpu.VMEM((CH_E + 2 * EBLK,), jnp.int32),
            pltpu.VMEM((LANES,), jnp.int32),
        ],
        compiler_params=pltpu.CompilerParams(use_tc_tiling_on_sc=False,
                                             needs_layout_passes=False),
    )
    chbm, cnts = f1(srcw, dstw, slotmap)

    f2 = pl.kernel(
        _reduce_body,
        out_type=jax.ShapeDtypeStruct((2, NCHUNK, SLOT_ROWS, LANES), jnp.float32),
        mesh=mesh,
        scratch_types=[
            pltpu.VMEM((STRIP * EBLK,), jnp.int32),
            pltpu.VMEM((STRIP, EBLK), jnp.int32),
            pltpu.VMEM((STRIP, EBLK), jnp.int32),
            pltpu.VMEM((STRIP, EBLK, LANES), jnp.float32),
            pltpu.VMEM((LANES,), jnp.int32),
            pltpu.VMEM((SROWS_PER_TILE, LANES), jnp.float32),
            pltpu.VMEM_SHARED((SLOT_ROWS, LANES), jnp.float32),
            pltpu.SemaphoreType.DMA,
            pltpu.SemaphoreType.DMA,
        ],
        compiler_params=pltpu.CompilerParams(use_tc_tiling_on_sc=False,
                                             needs_layout_passes=False),
    )
    return f2(x4, chbm, cnts, zrow)


def kernel(batch_data, edge_index, user_embedding, item_embedding, W_den, W_os,
           conv_w, last_stage, degree_new, degree_old):
    total_embeddings = jnp.concatenate([user_embedding, item_embedding], axis=0)

    x_den = jax.nn.relu(degree_old @ W_den.T) + degree_new
    degree_den = jnp.sqrt(x_den)
    norm_degree = (1.0 / (degree_den + 1e-9)).reshape(-1)

    users = batch_data[:, 0, 0]
    items = batch_data[:, 0, 1:]
    nodes = jnp.concatenate([users, (N_USERS + 1 + items).reshape(-1)])
    slotmap = jnp.full((MAP_ROWS,), TRASH_SLOT, jnp.int32)
    slotmap = slotmap.at[nodes].set(jnp.arange(NSLOT, dtype=jnp.int32))
    slots_b = slotmap[nodes]

    x1 = norm_degree[:, None] * total_embeddings
    x4 = x1.reshape(N_NODES * NCHUNK, LANES)

    dst = edge_index[0]
    src = edge_index[1]
    pad = E_PAD - N_EDGES
    srcw = jnp.concatenate([src, jnp.zeros((pad,), jnp.int32)]).reshape(NW, E_PER_W)
    dstw = jnp.concatenate([dst, jnp.full((pad,), DST_PAD, jnp.int32)]).reshape(NW, E_PER_W)
    zrow = jnp.zeros((SROWS_PER_TILE, LANES), jnp.float32)

    part = _gcn_sc(x4, srcw, dstw, slotmap, zrow)
    summed = (part[0] + part[1])[:, :NSLOT]
    emb1_slots = summed.transpose(1, 0, 2).reshape(NSLOT, EMB_DIM)

    rscale_vec = jnp.sqrt(jax.nn.relu(degree_old @ W_os.T)) / (degree_den + 1e-9)

    sc_nodes = jnp.stack([norm_degree, rscale_vec.reshape(-1)], axis=1)
    sc_b = sc_nodes[nodes]
    nd_b = sc_b[:, 0:1]
    rs_b = sc_b[:, 1:2]
    emb1_b = nd_b * emb1_slots[slots_b]
    te_b = total_embeddings[nodes]
    ls_b = jnp.take(last_stage, nodes, axis=1)

    fused0 = conv_w[0, 0] * (ls_b[0] * rs_b) + conv_w[0, 1] * te_b
    fused1 = conv_w[1, 0] * (ls_b[1] * rs_b) + conv_w[1, 1] * emb1_b
    layer_b = (te_b + fused0 + fused1) / 3.0
    nrm = jnp.linalg.norm(layer_b, axis=-1, keepdims=True)
    layer_b = layer_b / jnp.maximum(nrm, 1e-12)
    total2_b = layer_b + te_b

    user_feature = total2_b[:BATCH][:, None, :]
    item_feature = total2_b[BATCH:].reshape(BATCH, 2, EMB_DIM)
    scores = jnp.sum(user_feature * item_feature, axis=2)
    bpr = -jnp.mean(jax.nn.log_sigmoid(scores[:, 0] - scores[:, 1]))

    emb_loss = (jnp.linalg.norm(user_embedding) + jnp.linalg.norm(item_embedding)) / item_embedding.shape[0]
    return bpr + REG_WEIGHT * emb_loss


# speedup vs baseline: 3.5730x; 1.2485x over previous
import jax
import jax.numpy as jnp
from jax import lax
from jax.experimental import pallas as pl
from jax.experimental.pallas import tpu as pltpu
from jax.experimental.pallas import tpu_sc as plsc

N_USERS = 50000
N_ITEMS = 50000
N_NODES = N_USERS + 1 + N_ITEMS + 1
EMB_DIM = 64
N_EDGES = 1000000
BATCH = 4096
REG_WEIGHT = 1e-4

LANES = 16
NW = 32
EBLK = 128
E_PER_W = 31744
BLK_PER_W = E_PER_W // EBLK
E_PAD = NW * E_PER_W
CH_BLKS = 31
N_CH = BLK_PER_W // CH_BLKS
CH_E = CH_BLKS * EBLK
MAP_ROWS = 100352
NSLOT = 3 * BATCH
SLOT_ROWS = 12544
TRASH_SLOT = NSLOT
SROWS_PER_TILE = SLOT_ROWS // 16
DST_PAD = N_NODES
NCHUNK = 2
GLW = 32
STRIP = 8


def _filter_body(src_hbm, dst_hbm, map_hbm, chbm, cnt_hbm,
                 mp, dqc, sqc, cbuf, cst):
    ci = lax.axis_index("c")
    si = lax.axis_index("s")
    wid = si * 2 + ci
    tblk = wid * BLK_PER_W

    pltpu.sync_copy(map_hbm, mp)

    trash16 = jnp.full((LANES,), TRASH_SLOT, jnp.int32)
    nflushed = jnp.int32(0)
    for ch in range(N_CH):
        pltpu.sync_copy(dst_hbm.at[wid, pl.ds(ch * CH_E, CH_E)], dqc)
        pltpu.sync_copy(src_hbm.at[wid, pl.ds(ch * CH_E, CH_E)], sqc)

        def blk(j, cur):
            for g in range(EBLK // LANES):
                d16 = dqc[pl.ds(j * EBLK + g * LANES, LANES)]
                s16 = sqc[pl.ds(j * EBLK + g * LANES, LANES)]
                slot16 = plsc.load_gather(mp, [d16])
                m = slot16 != TRASH_SLOT
                packed = s16 * 16384 + slot16
                plsc.store_compressed(cbuf.at[pl.ds(cur, LANES)], packed, mask=m)
                cur = cur + jnp.sum(m.astype(jnp.int32))
            return cur

        cur = lax.fori_loop(0, CH_BLKS, blk, jnp.int32(0))
        tgt = ((cur + EBLK - 1) // EBLK) * EBLK
        for g in range(EBLK // LANES):
            @pl.when(cur + g * LANES < tgt)
            def _():
                cbuf[pl.ds(cur + g * LANES, LANES)] = trash16

        nb = tgt // EBLK

        def flush(b, _):
            pltpu.sync_copy(cbuf.at[pl.ds(b * EBLK, EBLK)],
                            chbm.at[pl.ds((tblk + nflushed + b) * EBLK, EBLK)])
            return _

        lax.fori_loop(0, nb, flush, jnp.int32(0))
        nflushed = nflushed + nb

    for g in range(EBLK // LANES):
        cbuf[pl.ds(g * LANES, LANES)] = trash16
    tpad = ((nflushed + STRIP - 1) // STRIP) * STRIP

    def padb(b, _):
        pltpu.sync_copy(cbuf.at[pl.ds(0, EBLK)],
                        chbm.at[pl.ds((tblk + nflushed + b) * EBLK, EBLK)])
        return _

    lax.fori_loop(0, tpad - nflushed, padb, jnp.int32(0))
    cst[pl.ds(0, LANES)] = jnp.full((LANES,), 1, jnp.int32) * tpad
    pltpu.sync_copy(cst, cnt_hbm.at[pl.ds(wid * LANES, LANES)])


def _reduce_body(x4_hbm, chbm, cnt_hbm, zrow_hbm, out_hbm,
                 pbl, gist, sst, rows, cntv, zdbuf, acc, sem, sem2):
    ci = lax.axis_index("c")
    si = lax.axis_index("s")
    wid = si * 2 + ci
    tblk = wid * BLK_PER_W

    pltpu.sync_copy(cnt_hbm.at[pl.ds(wid * LANES, LANES)], cntv)
    nblocks = cntv[pl.ds(0, LANES)][0]
    nstrips = nblocks // STRIP

    for c in range(NCHUNK):
        pltpu.sync_copy(zrow_hbm, zdbuf)
        pltpu.sync_copy(zdbuf, acc.at[pl.ds(si * SROWS_PER_TILE, SROWS_PER_TILE)])
        plsc.subcore_barrier()

        def strip_fn(st, _):
            base = (tblk + st * STRIP) * EBLK
            pltpu.sync_copy(chbm.at[pl.ds(base, STRIP * EBLK)], pbl)
            for j in range(STRIP):
                for g in range(EBLK // LANES):
                    p16 = pbl[pl.ds(j * EBLK + g * LANES, LANES)]
                    slot16 = jnp.bitwise_and(p16, 16383)
                    src16 = lax.shift_right_logical(p16, 14)
                    gist[j, pl.ds(g * LANES, LANES)] = src16 * NCHUNK + c
                    sst[j, pl.ds(g * LANES, LANES)] = slot16
            for lo in range(0, STRIP, 8):
                hi = min(lo + 8, STRIP)
                gh = [pltpu.async_copy(x4_hbm.at[gist.at[j]], rows.at[j], sem)
                      for j in range(lo, hi)]
                sh = []
                for k, j in enumerate(range(lo, hi)):
                    gh[k].wait()
                    sh.append(pltpu.async_copy(rows.at[j], acc.at[sst.at[j]],
                                               sem2, add=True))
                for h in sh:
                    h.wait()
            return _

        lax.fori_loop(0, nstrips, strip_fn, jnp.int32(0))
        plsc.subcore_barrier()

        r = si * SROWS_PER_TILE
        pltpu.sync_copy(acc.at[pl.ds(r, SROWS_PER_TILE)], zdbuf)
        pltpu.sync_copy(zdbuf, out_hbm.at[ci, c, pl.ds(r, SROWS_PER_TILE)])
        plsc.subcore_barrier()


@jax.jit
def _gcn_sc(x4, srcw, dstw, slotmap, zrow):
    mesh = plsc.VectorSubcoreMesh(core_axis_name="c", subcore_axis_name="s")
    f1 = pl.kernel(
        _filter_body,
        out_type=(
            jax.ShapeDtypeStruct((E_PAD,), jnp.int32),
            jax.ShapeDtypeStruct((NW * LANES,), jnp.int32),
        ),
        mesh=mesh,
        scratch_types=[
            pltpu.VMEM((MAP_ROWS,), jnp.int32),
            pltpu.VMEM((CH_E,), jnp.int32),
            pltpu.VMEM((CH_E,), jnp.int32),
            pltpu.VMEM((CH_E + 2 * EBLK,), jnp.int32),
            pltpu.VMEM((LANES,), jnp.int32),
        ],
        compiler_params=pltpu.CompilerParams(use_tc_tiling_on_sc=False,
                                             needs_layout_passes=False),
    )
    chbm, cnts = f1(srcw, dstw, slotmap)

    f2 = pl.kernel(
        _reduce_body,
        out_type=jax.ShapeDtypeStruct((2, NCHUNK, SLOT_ROWS, GLW), jnp.float32),
        mesh=mesh,
        scratch_types=[
            pltpu.VMEM((STRIP * EBLK,), jnp.int32),
            pltpu.VMEM((STRIP, EBLK), jnp.int32),
            pltpu.VMEM((STRIP, EBLK), jnp.int32),
            pltpu.VMEM((STRIP, EBLK, GLW), jnp.float32),
            pltpu.VMEM((LANES,), jnp.int32),
            pltpu.VMEM((SROWS_PER_TILE, GLW), jnp.float32),
            pltpu.VMEM_SHARED((SLOT_ROWS, GLW), jnp.float32),
            pltpu.SemaphoreType.DMA,
            pltpu.SemaphoreType.DMA,
        ],
        compiler_params=pltpu.CompilerParams(use_tc_tiling_on_sc=False,
                                             needs_layout_passes=False),
    )
    return f2(x4, chbm, cnts, zrow)


def kernel(batch_data, edge_index, user_embedding, item_embedding, W_den, W_os,
           conv_w, last_stage, degree_new, degree_old):
    total_embeddings = jnp.concatenate([user_embedding, item_embedding], axis=0)

    x_den = jax.nn.relu(degree_old @ W_den.T) + degree_new
    degree_den = jnp.sqrt(x_den)
    norm_degree = (1.0 / (degree_den + 1e-9)).reshape(-1)

    users = batch_data[:, 0, 0]
    items = batch_data[:, 0, 1:]
    nodes = jnp.concatenate([users, (N_USERS + 1 + items).reshape(-1)])
    slotmap = jnp.full((MAP_ROWS,), TRASH_SLOT, jnp.int32)
    slotmap = slotmap.at[nodes].set(jnp.arange(NSLOT, dtype=jnp.int32))
    slots_b = slotmap[nodes]

    x1 = norm_degree[:, None] * total_embeddings
    x4 = x1.reshape(N_NODES * NCHUNK, GLW)

    dst = edge_index[0]
    src = edge_index[1]
    pad = E_PAD - N_EDGES
    srcw = jnp.concatenate([src, jnp.zeros((pad,), jnp.int32)]).reshape(NW, E_PER_W)
    dstw = jnp.concatenate([dst, jnp.full((pad,), DST_PAD, jnp.int32)]).reshape(NW, E_PER_W)
    zrow = jnp.zeros((SROWS_PER_TILE, GLW), jnp.float32)

    part = _gcn_sc(x4, srcw, dstw, slotmap, zrow)
    summed = (part[0] + part[1])[:, :NSLOT]
    emb1_slots = summed.transpose(1, 0, 2).reshape(NSLOT, EMB_DIM)

    rscale_vec = jnp.sqrt(jax.nn.relu(degree_old @ W_os.T)) / (degree_den + 1e-9)

    sc_nodes = jnp.stack([norm_degree, rscale_vec.reshape(-1)], axis=1)
    sc_b = sc_nodes[nodes]
    nd_b = sc_b[:, 0:1]
    rs_b = sc_b[:, 1:2]
    emb1_b = nd_b * emb1_slots[slots_b]
    te_b = total_embeddings[nodes]
    ls_b = jnp.take(last_stage, nodes, axis=1)

    fused0 = conv_w[0, 0] * (ls_b[0] * rs_b) + conv_w[0, 1] * te_b
    fused1 = conv_w[1, 0] * (ls_b[1] * rs_b) + conv_w[1, 1] * emb1_b
    layer_b = (te_b + fused0 + fused1) / 3.0
    nrm = jnp.linalg.norm(layer_b, axis=-1, keepdims=True)
    layer_b = layer_b / jnp.maximum(nrm, 1e-12)
    total2_b = layer_b + te_b

    user_feature = total2_b[:BATCH][:, None, :]
    item_feature = total2_b[BATCH:].reshape(BATCH, 2, EMB_DIM)
    scores = jnp.sum(user_feature * item_feature, axis=2)
    bpr = -jnp.mean(jax.nn.log_sigmoid(scores[:, 0] - scores[:, 1]))

    emb_loss = (jnp.linalg.norm(user_embedding) + jnp.linalg.norm(item_embedding)) / item_embedding.shape[0]
    return bpr + REG_WEIGHT * emb_loss


# speedup vs baseline: 3.5784x; 1.0015x over previous
import jax
import jax.numpy as jnp
from jax import lax
from jax.experimental import pallas as pl
from jax.experimental.pallas import tpu as pltpu
from jax.experimental.pallas import tpu_sc as plsc

N_USERS = 50000
N_ITEMS = 50000
N_NODES = N_USERS + 1 + N_ITEMS + 1
EMB_DIM = 64
N_EDGES = 1000000
BATCH = 4096
REG_WEIGHT = 1e-4

LANES = 16
NW = 32
EBLK = 128
E_PER_W = 31744
BLK_PER_W = E_PER_W // EBLK
E_PAD = NW * E_PER_W
CH_BLKS = 31
N_CH = BLK_PER_W // CH_BLKS
CH_E = CH_BLKS * EBLK
MAP_ROWS = 100352
NSLOT = 3 * BATCH
SLOT_ROWS = 12544
TRASH_SLOT = NSLOT
SROWS_PER_TILE = SLOT_ROWS // 16
DST_PAD = N_NODES
NCHUNK = 2
GLW = 32
STRIP = 8


def _filter_body(src_hbm, dst_hbm, map_hbm, chbm, cnt_hbm,
                 mp, dqc, sqc, cbuf, cst):
    ci = lax.axis_index("c")
    si = lax.axis_index("s")
    wid = si * 2 + ci
    tblk = wid * BLK_PER_W

    pltpu.sync_copy(map_hbm, mp)

    trash16 = jnp.full((LANES,), TRASH_SLOT, jnp.int32)
    nflushed = jnp.int32(0)
    for ch in range(N_CH):
        pltpu.sync_copy(dst_hbm.at[wid, pl.ds(ch * CH_E, CH_E)], dqc)
        pltpu.sync_copy(src_hbm.at[wid, pl.ds(ch * CH_E, CH_E)], sqc)

        def blk(j, cur):
            for g in range(EBLK // LANES):
                d16 = dqc[pl.ds(j * EBLK + g * LANES, LANES)]
                s16 = sqc[pl.ds(j * EBLK + g * LANES, LANES)]
                slot16 = plsc.load_gather(mp, [d16])
                m = slot16 != TRASH_SLOT
                packed = s16 * 16384 + slot16
                plsc.store_compressed(cbuf.at[pl.ds(cur, LANES)], packed, mask=m)
                cur = cur + jnp.sum(m.astype(jnp.int32))
            return cur

        cur = lax.fori_loop(0, CH_BLKS, blk, jnp.int32(0))
        tgt = ((cur + EBLK - 1) // EBLK) * EBLK
        for g in range(EBLK // LANES):
            @pl.when(cur + g * LANES < tgt)
            def _():
                cbuf[pl.ds(cur + g * LANES, LANES)] = trash16

        nb = tgt // EBLK

        def flush(b, _):
            pltpu.sync_copy(cbuf.at[pl.ds(b * EBLK, EBLK)],
                            chbm.at[pl.ds((tblk + nflushed + b) * EBLK, EBLK)])
            return _

        lax.fori_loop(0, nb, flush, jnp.int32(0))
        nflushed = nflushed + nb

    for g in range(EBLK // LANES):
        cbuf[pl.ds(g * LANES, LANES)] = trash16
    tpad = ((nflushed + STRIP - 1) // STRIP) * STRIP

    def padb(b, _):
        pltpu.sync_copy(cbuf.at[pl.ds(0, EBLK)],
                        chbm.at[pl.ds((tblk + nflushed + b) * EBLK, EBLK)])
        return _

    lax.fori_loop(0, tpad - nflushed, padb, jnp.int32(0))
    cst[pl.ds(0, LANES)] = jnp.full((LANES,), 1, jnp.int32) * tpad
    pltpu.sync_copy(cst, cnt_hbm.at[pl.ds(wid * LANES, LANES)])


def _reduce_body(x4_hbm, chbm, cnt_hbm, zrow_hbm, out_hbm,
                 pbl, gist, sst, rows, cntv, zdbuf, acc, sem, sem2):
    ci = lax.axis_index("c")
    si = lax.axis_index("s")
    wid = si * 2 + ci
    tblk = wid * BLK_PER_W

    pltpu.sync_copy(cnt_hbm.at[pl.ds(wid * LANES, LANES)], cntv)
    nblocks = cntv[pl.ds(0, LANES)][0]
    nstrips = nblocks // STRIP

    for c in range(NCHUNK):
        pltpu.sync_copy(zrow_hbm, acc.at[pl.ds(si * SROWS_PER_TILE, SROWS_PER_TILE)])
        plsc.subcore_barrier()

        def strip_fn(st, _):
            base = (tblk + st * STRIP) * EBLK
            pltpu.sync_copy(chbm.at[pl.ds(base, STRIP * EBLK)], pbl)
            for j in range(STRIP):
                for g in range(EBLK // LANES):
                    p16 = pbl[pl.ds(j * EBLK + g * LANES, LANES)]
                    slot16 = jnp.bitwise_and(p16, 16383)
                    src16 = lax.shift_right_logical(p16, 14)
                    gist[j, pl.ds(g * LANES, LANES)] = src16 * NCHUNK + c
                    sst[j, pl.ds(g * LANES, LANES)] = slot16
            for lo in range(0, STRIP, 8):
                hi = min(lo + 8, STRIP)
                gh = [pltpu.async_copy(x4_hbm.at[gist.at[j]], rows.at[j], sem)
                      for j in range(lo, hi)]
                sh = []
                for k, j in enumerate(range(lo, hi)):
                    gh[k].wait()
                    sh.append(pltpu.async_copy(rows.at[j], acc.at[sst.at[j]],
                                               sem2, add=True))
                for h in sh:
                    h.wait()
            return _

        lax.fori_loop(0, nstrips, strip_fn, jnp.int32(0))
        plsc.subcore_barrier()

        r = si * SROWS_PER_TILE
        pltpu.sync_copy(acc.at[pl.ds(r, SROWS_PER_TILE)],
                        out_hbm.at[ci, c, pl.ds(r, SROWS_PER_TILE)])
        plsc.subcore_barrier()


@jax.jit
def _gcn_sc(x4, srcw, dstw, slotmap, zrow):
    mesh = plsc.VectorSubcoreMesh(core_axis_name="c", subcore_axis_name="s")
    f1 = pl.kernel(
        _filter_body,
        out_type=(
            jax.ShapeDtypeStruct((E_PAD,), jnp.int32),
            jax.ShapeDtypeStruct((NW * LANES,), jnp.int32),
        ),
        mesh=mesh,
        scratch_types=[
            pltpu.VMEM((MAP_ROWS,), jnp.int32),
            pltpu.VMEM((CH_E,), jnp.int32),
            pltpu.VMEM((CH_E,), jnp.int32),
            pltpu.VMEM((CH_E + 2 * EBLK,), jnp.int32),
            pltpu.VMEM((LANES,), jnp.int32),
        ],
        compiler_params=pltpu.CompilerParams(use_tc_tiling_on_sc=False,
                                             needs_layout_passes=False),
    )
    chbm, cnts = f1(srcw, dstw, slotmap)

    f2 = pl.kernel(
        _reduce_body,
        out_type=jax.ShapeDtypeStruct((2, NCHUNK, SLOT_ROWS, GLW), jnp.float32),
        mesh=mesh,
        scratch_types=[
            pltpu.VMEM((STRIP * EBLK,), jnp.int32),
            pltpu.VMEM((STRIP, EBLK), jnp.int32),
            pltpu.VMEM((STRIP, EBLK), jnp.int32),
            pltpu.VMEM((STRIP, EBLK, GLW), jnp.float32),
            pltpu.VMEM((LANES,), jnp.int32),
            pltpu.VMEM((SROWS_PER_TILE, GLW), jnp.float32),
            pltpu.VMEM_SHARED((SLOT_ROWS, GLW), jnp.float32),
            pltpu.SemaphoreType.DMA,
            pltpu.SemaphoreType.DMA,
        ],
        compiler_params=pltpu.CompilerParams(use_tc_tiling_on_sc=False,
                                             needs_layout_passes=False),
    )
    return f2(x4, chbm, cnts, zrow)


def kernel(batch_data, edge_index, user_embedding, item_embedding, W_den, W_os,
           conv_w, last_stage, degree_new, degree_old):
    total_embeddings = jnp.concatenate([user_embedding, item_embedding], axis=0)

    x_den = jax.nn.relu(degree_old @ W_den.T) + degree_new
    degree_den = jnp.sqrt(x_den)
    norm_degree = (1.0 / (degree_den + 1e-9)).reshape(-1)

    users = batch_data[:, 0, 0]
    items = batch_data[:, 0, 1:]
    nodes = jnp.concatenate([users, (N_USERS + 1 + items).reshape(-1)])
    slotmap = jnp.full((MAP_ROWS,), TRASH_SLOT, jnp.int32)
    slotmap = slotmap.at[nodes].set(jnp.arange(NSLOT, dtype=jnp.int32))
    slots_b = slotmap[nodes]

    x1 = norm_degree[:, None] * total_embeddings
    x4 = x1.reshape(N_NODES * NCHUNK, GLW)

    dst = edge_index[0]
    src = edge_index[1]
    pad = E_PAD - N_EDGES
    srcw = jnp.concatenate([src, jnp.zeros((pad,), jnp.int32)]).reshape(NW, E_PER_W)
    dstw = jnp.concatenate([dst, jnp.full((pad,), DST_PAD, jnp.int32)]).reshape(NW, E_PER_W)
    zrow = jnp.zeros((SROWS_PER_TILE, GLW), jnp.float32)

    part = _gcn_sc(x4, srcw, dstw, slotmap, zrow)
    summed = (part[0] + part[1])[:, :NSLOT]
    emb1_slots = summed.transpose(1, 0, 2).reshape(NSLOT, EMB_DIM)

    rscale_vec = jnp.sqrt(jax.nn.relu(degree_old @ W_os.T)) / (degree_den + 1e-9)

    sc_nodes = jnp.stack([norm_degree, rscale_vec.reshape(-1)], axis=1)
    sc_b = sc_nodes[nodes]
    nd_b = sc_b[:, 0:1]
    rs_b = sc_b[:, 1:2]
    emb1_b = nd_b * emb1_slots[slots_b]
    te_b = total_embeddings[nodes]
    ls_b = jnp.take(last_stage, nodes, axis=1)

    fused0 = conv_w[0, 0] * (ls_b[0] * rs_b) + conv_w[0, 1] * te_b
    fused1 = conv_w[1, 0] * (ls_b[1] * rs_b) + conv_w[1, 1] * emb1_b
    layer_b = (te_b + fused0 + fused1) / 3.0
    nrm = jnp.linalg.norm(layer_b, axis=-1, keepdims=True)
    layer_b = layer_b / jnp.maximum(nrm, 1e-12)
    total2_b = layer_b + te_b

    user_feature = total2_b[:BATCH][:, None, :]
    item_feature = total2_b[BATCH:].reshape(BATCH, 2, EMB_DIM)
    scores = jnp.sum(user_feature * item_feature, axis=2)
    bpr = -jnp.mean(jax.nn.log_sigmoid(scores[:, 0] - scores[:, 1]))

    emb_loss = (jnp.linalg.norm(user_embedding) + jnp.linalg.norm(item_embedding)) / item_embedding.shape[0]
    return bpr + REG_WEIGHT * emb_loss
